# Initial kernel scaffold; baseline (speedup 1.0000x reference)
import jax
import jax.numpy as jnp
from jax import lax
from jax.experimental import pallas as pl
from jax.experimental.pallas import tpu as pltpu
from jax.experimental.pallas import tpu_sc as plsc

L = 16
NC = 2
NS = 16
NW = NC * NS

B, SEQ = 16384, 200
N = B * SEQ
VOCAB, D = 100, 200
KPAD = 112
PER_W = N // NW
CHUNK = 25600
NCHUNK = PER_W // CHUNK


def _lut_gather_body(xf, embt, wb, out, et_v, wb_v, p_v, idx_v, out_v):
    wid = lax.axis_index("s") * NC + lax.axis_index("c")

    pltpu.sync_copy(embt, et_v)
    pltpu.sync_copy(wb, wb_v)
    bias = wb_v[D]

    for g in range(KPAD // L):
        def dstep(d, acc):
            return acc + et_v[d, pl.ds(g * L, L)] * wb_v[d]
        p_v[pl.ds(g * L, L)] = lax.fori_loop(
            0, D, dstep, jnp.full((L,), bias, jnp.float32))

    for c in range(NCHUNK):
        off = wid * PER_W + c * CHUNK
        pltpu.sync_copy(xf.at[pl.ds(off, CHUNK)], idx_v)

        def gstep(i, carry):
            sl = pl.ds(i * L, L)
            out_v[sl] = plsc.load_gather(p_v, [idx_v[sl]])
            return carry
        lax.fori_loop(0, CHUNK // L, gstep, 0)
        pltpu.sync_copy(out_v, out.at[pl.ds(off, CHUNK)])


def kernel(x, emb, fc_w, fc_b):
    xf = x.reshape(-1)
    embt = jnp.pad(emb.T, ((0, 0), (0, KPAD - VOCAB)))
    wb = jnp.pad(jnp.concatenate([fc_w.reshape(-1), fc_b]), (0, 7))
    run = pl.kernel(
        _lut_gather_body,
        out_type=jax.ShapeDtypeStruct((N,), jnp.float32),
        mesh=plsc.VectorSubcoreMesh(core_axis_name="c", subcore_axis_name="s"),
        scratch_types=[
            pltpu.VMEM((D, KPAD), jnp.float32),
            pltpu.VMEM((D + 8,), jnp.float32),
            pltpu.VMEM((KPAD,), jnp.float32),
            pltpu.VMEM((CHUNK,), jnp.int32),
            pltpu.VMEM((CHUNK,), jnp.float32),
        ],
    )
    return run(xf, embt, wb).reshape(B, SEQ, 1)


# speedup vs baseline: 89.6606x; 89.6606x over previous
import jax
import jax.numpy as jnp
from jax import lax
from jax.experimental import pallas as pl
from jax.experimental.pallas import tpu as pltpu
from jax.experimental.pallas import tpu_sc as plsc

L = 16
NC = 2
NS = 16
NW = NC * NS

B, SEQ = 16384, 200
N = B * SEQ
VOCAB, D = 100, 200
KPAD = 112
PER_W = N // NW
CHUNK = 25600
NCHUNK = PER_W // CHUNK


def _lut_gather_body(xf, embt, wbc, out, et_v, wb_v, p_v, idx_v, out_v):
    wid = lax.axis_index("s") * NC + lax.axis_index("c")

    pltpu.sync_copy(embt, et_v)
    pltpu.sync_copy(wbc, wb_v)

    for g in range(KPAD // L):
        def dstep(d, acc):
            return acc + et_v[d, pl.ds(g * L, L)] * wb_v[d, :]
        p_v[pl.ds(g * L, L)] = lax.fori_loop(0, D, dstep, wb_v[D, :])

    for c in range(NCHUNK):
        off = wid * PER_W + c * CHUNK
        pltpu.sync_copy(xf.at[pl.ds(off, CHUNK)], idx_v)

        def gstep(i, carry):
            sl = pl.ds(i * L, L)
            out_v[sl] = plsc.load_gather(p_v, [idx_v[sl]])
            return carry
        lax.fori_loop(0, CHUNK // L, gstep, 0)
        pltpu.sync_copy(out_v, out.at[pl.ds(off, CHUNK)])


def kernel(x, emb, fc_w, fc_b):
    xf = x.reshape(-1)
    embt = jnp.pad(emb.T, ((0, 0), (0, KPAD - VOCAB)))
    wbc = jnp.broadcast_to(
        jnp.concatenate([fc_w.reshape(-1), fc_b])[:, None], (D + 1, L))
    run = pl.kernel(
        _lut_gather_body,
        out_type=jax.ShapeDtypeStruct((N,), jnp.float32),
        mesh=plsc.VectorSubcoreMesh(core_axis_name="c", subcore_axis_name="s"),
        compiler_params=pltpu.CompilerParams(needs_layout_passes=False),
        scratch_types=[
            pltpu.VMEM((D, KPAD), jnp.float32),
            pltpu.VMEM((D + 1, L), jnp.float32),
            pltpu.VMEM((KPAD,), jnp.float32),
            pltpu.VMEM((CHUNK,), jnp.int32),
            pltpu.VMEM((CHUNK,), jnp.float32),
        ],
    )
    return run(xf, embt, wbc).reshape(B, SEQ, 1)
